# Initial kernel scaffold; baseline (speedup 1.0000x reference)
import jax
import jax.numpy as jnp
from jax.experimental import pallas as pl


def kernel(nodes, edge_index, W_enc, b_enc, W_root, W_neigh, b_conv, W_pred, b_pred):
    raise NotImplementedError("write your pallas kernel here")


# speedup vs baseline: 3.4596x; 3.4596x over previous
import jax
import jax.numpy as jnp
from jax import lax
from jax.experimental import pallas as pl
from jax.experimental.pallas import tpu as pltpu
from jax.experimental.pallas import tpu_sc as plsc

N = 10000
E = 320000
D = 128
D2 = D // 2

NC = 2
NS = 16
EPT = E // NS
CH = 80
NCHT = EPT // CH
NPAD = 10240
RPS = NPAD // NS
DEGW = 16
ZR = 160
NZ = RPS // ZR

_MESH = plsc.VectorSubcoreMesh(core_axis_name="c", subcore_axis_name="s")


def _make_sc_layer(with_deg):
    out_type = [jax.ShapeDtypeStruct((NC, NPAD, D2), jnp.float32)]
    scratch = [
        pltpu.VMEM_SHARED((NPAD, D2), jnp.float32),
        pltpu.VMEM((CH,), jnp.int32),
        pltpu.VMEM((CH,), jnp.int32),
        pltpu.VMEM((CH, D2), jnp.float32),
        pltpu.VMEM((ZR, D2), jnp.float32),
        pltpu.SemaphoreType.DMA,
    ]
    if with_deg:
        out_type.append(jax.ShapeDtypeStruct((NPAD, DEGW), jnp.float32))
        scratch.insert(1, pltpu.VMEM_SHARED((NPAD, DEGW), jnp.float32))
        scratch.append(pltpu.VMEM((CH, DEGW), jnp.float32))
        scratch.append(pltpu.VMEM((ZR, DEGW), jnp.float32))

    def body_deg(hlo_hbm, hhi_hbm, src_hbm, dst_hbm, z64_hbm, z16_hbm,
                 ones_hbm,
                 agg_out, deg_out,
                 agg_sh, deg_sh, src_v, dst_v, rows_v, stage_v, sem,
                 ones_v, dstage_v):
        cid = lax.axis_index("c")
        sid = lax.axis_index("s")
        row0 = sid * RPS
        pltpu.sync_copy(z64_hbm, stage_v)
        pltpu.sync_copy(z16_hbm, dstage_v)
        pltpu.sync_copy(ones_hbm, ones_v)

        def zstep(k, carry):
            r = row0 + k * ZR
            pltpu.sync_copy(stage_v, agg_sh.at[pl.ds(r, ZR)])
            pltpu.sync_copy(dstage_v, deg_sh.at[pl.ds(r, ZR)])
            return carry

        lax.fori_loop(0, NZ, zstep, 0)
        plsc.subcore_barrier()

        def make_step(h_half, add_deg):
            def step(c, carry):
                base = sid * EPT + c * CH
                pltpu.sync_copy(src_hbm.at[pl.ds(base, CH)], src_v)
                pltpu.sync_copy(dst_hbm.at[pl.ds(base, CH)], dst_v)
                pltpu.async_copy(h_half.at[src_v], rows_v, sem).wait()
                pltpu.sync_copy(rows_v, agg_sh.at[dst_v], add=True)
                if add_deg:
                    pltpu.sync_copy(ones_v, deg_sh.at[dst_v], add=True)
                return carry
            return step

        @pl.when(cid == 0)
        def _():
            lax.fori_loop(0, NCHT, make_step(hlo_hbm, True), 0)

        @pl.when(cid == 1)
        def _():
            lax.fori_loop(0, NCHT, make_step(hhi_hbm, False), 0)

        plsc.subcore_barrier()

        def wstep(k, carry):
            r = row0 + k * ZR
            pltpu.sync_copy(agg_sh.at[pl.ds(r, ZR)], stage_v)
            pltpu.sync_copy(stage_v, agg_out.at[cid, pl.ds(r, ZR)])
            return carry

        lax.fori_loop(0, NZ, wstep, 0)

        @pl.when(cid == 0)
        def _():
            def dwstep(k, carry):
                r = row0 + k * ZR
                pltpu.sync_copy(deg_sh.at[pl.ds(r, ZR)], dstage_v)
                pltpu.sync_copy(dstage_v, deg_out.at[pl.ds(r, ZR)])
                return carry
            lax.fori_loop(0, NZ, dwstep, 0)

    def body_nodeg(hlo_hbm, hhi_hbm, src_hbm, dst_hbm, z64_hbm,
                   agg_out,
                   agg_sh, src_v, dst_v, rows_v, stage_v, sem):
        cid = lax.axis_index("c")
        sid = lax.axis_index("s")
        row0 = sid * RPS
        pltpu.sync_copy(z64_hbm, stage_v)

        def zstep(k, carry):
            pltpu.sync_copy(stage_v, agg_sh.at[pl.ds(row0 + k * ZR, ZR)])
            return carry

        lax.fori_loop(0, NZ, zstep, 0)
        plsc.subcore_barrier()

        def make_step(h_half):
            def step(c, carry):
                base = sid * EPT + c * CH
                pltpu.sync_copy(src_hbm.at[pl.ds(base, CH)], src_v)
                pltpu.sync_copy(dst_hbm.at[pl.ds(base, CH)], dst_v)
                pltpu.async_copy(h_half.at[src_v], rows_v, sem).wait()
                pltpu.sync_copy(rows_v, agg_sh.at[dst_v], add=True)
                return carry
            return step

        @pl.when(cid == 0)
        def _():
            lax.fori_loop(0, NCHT, make_step(hlo_hbm), 0)

        @pl.when(cid == 1)
        def _():
            lax.fori_loop(0, NCHT, make_step(hhi_hbm), 0)

        plsc.subcore_barrier()

        def wstep(k, carry):
            r = row0 + k * ZR
            pltpu.sync_copy(agg_sh.at[pl.ds(r, ZR)], stage_v)
            pltpu.sync_copy(stage_v, agg_out.at[cid, pl.ds(r, ZR)])
            return carry

        lax.fori_loop(0, NZ, wstep, 0)

    body = body_deg if with_deg else body_nodeg
    return pl.kernel(
        body, out_type=tuple(out_type), mesh=_MESH,
        scratch_types=tuple(scratch),
        compiler_params=pltpu.CompilerParams(use_tc_tiling_on_sc=False))


_sc_layer_deg = _make_sc_layer(True)
_sc_layer = _make_sc_layer(False)


BR = 1000
G = N // BR


def _enc_body(x_ref, w_ref, b_ref, olo_ref, ohi_ref):
    z = (jnp.dot(x_ref[...], w_ref[...],
                 preferred_element_type=jnp.float32) + b_ref[...])
    olo_ref[...] = z[:, :D2]
    ohi_ref[...] = z[:, D2:]


def _layer_body(agg_ref, deg_ref, hlo_ref, hhi_ref, wn_ref, wr_ref, b_ref,
                olo_ref, ohi_ref):
    agg = jnp.concatenate([agg_ref[0], agg_ref[1]], axis=1)
    h = jnp.concatenate([hlo_ref[...], hhi_ref[...]], axis=1)
    inv = 1.0 / jnp.maximum(deg_ref[:, 0:1], 1.0)
    z = (jnp.dot(agg * inv, wn_ref[...], preferred_element_type=jnp.float32)
         + jnp.dot(h, wr_ref[...], preferred_element_type=jnp.float32)
         + b_ref[...])
    z = jnp.maximum(z, 0.0)
    olo_ref[...] = z[:, :D2]
    ohi_ref[...] = z[:, D2:]


def _layer_pred_body(agg_ref, deg_ref, hlo_ref, hhi_ref, wn_ref, wr_ref,
                     b_ref, wp_ref, bp_ref, o_ref):
    agg = jnp.concatenate([agg_ref[0], agg_ref[1]], axis=1)
    h = jnp.concatenate([hlo_ref[...], hhi_ref[...]], axis=1)
    inv = 1.0 / jnp.maximum(deg_ref[:, 0:1], 1.0)
    z = (jnp.dot(agg * inv, wn_ref[...], preferred_element_type=jnp.float32)
         + jnp.dot(h, wr_ref[...], preferred_element_type=jnp.float32)
         + b_ref[...])
    z = jnp.maximum(z, 0.0)
    o_ref[...] = (jnp.dot(z, wp_ref[...], preferred_element_type=jnp.float32)
                  + bp_ref[...])


_row_spec = pl.BlockSpec((BR, D), lambda i: (i, 0))
_half_spec = pl.BlockSpec((BR, D2), lambda i: (i, 0))
_w_spec = pl.BlockSpec((D, D), lambda i: (0, 0))
_b_spec = pl.BlockSpec((1, D), lambda i: (0, 0))
_agg_spec = pl.BlockSpec((NC, BR, D2), lambda i: (0, i, 0))
_deg_spec = pl.BlockSpec((BR, DEGW), lambda i: (i, 0))
_half_sds = jax.ShapeDtypeStruct((N, D2), jnp.float32)

_encoder = pl.pallas_call(
    _enc_body, grid=(G,),
    in_specs=[_row_spec, _w_spec, _b_spec],
    out_specs=[_half_spec, _half_spec], out_shape=[_half_sds, _half_sds])

_tc_layer = pl.pallas_call(
    _layer_body, grid=(G,),
    in_specs=[_agg_spec, _deg_spec, _half_spec, _half_spec, _w_spec, _w_spec,
              _b_spec],
    out_specs=[_half_spec, _half_spec], out_shape=[_half_sds, _half_sds])

_tc_layer_pred = pl.pallas_call(
    _layer_pred_body, grid=(G,),
    in_specs=[_agg_spec, _deg_spec, _half_spec, _half_spec, _w_spec, _w_spec,
              _b_spec, _w_spec, _b_spec],
    out_specs=_row_spec,
    out_shape=jax.ShapeDtypeStruct((N, D), jnp.float32))


def kernel(nodes, edge_index, W_enc, b_enc, W_root, W_neigh, b_conv,
           W_pred, b_pred):
    src = edge_index[0]
    dst = edge_index[1]
    z64 = jnp.zeros((ZR, D2), jnp.float32)
    z16 = jnp.zeros((ZR, DEGW), jnp.float32)
    ones_ch = jnp.ones((CH, DEGW), jnp.float32)

    h0lo, h0hi = _encoder(nodes, W_enc, b_enc.reshape(1, D))
    agg1, deg = _sc_layer_deg(h0lo, h0hi, src, dst, z64, z16, ones_ch)
    h1lo, h1hi = _tc_layer(agg1, deg, h0lo, h0hi, W_neigh[0], W_root[0],
                           b_conv[0].reshape(1, D))
    (agg2,) = _sc_layer(h1lo, h1hi, src, dst, z64)
    return _tc_layer_pred(agg2, deg, h1lo, h1hi, W_neigh[1], W_root[1],
                          b_conv[1].reshape(1, D), W_pred,
                          b_pred.reshape(1, D))


# speedup vs baseline: 9.0220x; 2.6078x over previous
import jax
import jax.numpy as jnp
from jax import lax
from jax.experimental import pallas as pl
from jax.experimental.pallas import tpu as pltpu
from jax.experimental.pallas import tpu_sc as plsc

N = 10000
E = 320000
D = 128
D2 = D // 2

NC = 2
NS = 16
EPT = E // NS
CH = 80
NCHT = EPT // CH
NPAD = 10240
RPS = NPAD // NS
DEGW = 16
ZR = 160
NZ = RPS // ZR

_MESH = plsc.VectorSubcoreMesh(core_axis_name="c", subcore_axis_name="s")


def _make_sc_layer(with_deg):
    out_type = [jax.ShapeDtypeStruct((NC, NPAD, D2), jnp.float32)]
    scratch = [
        pltpu.VMEM_SHARED((NPAD, D2), jnp.float32),
        pltpu.VMEM((NCHT, CH), jnp.int32),
        pltpu.VMEM((NCHT, CH), jnp.int32),
        pltpu.VMEM((CH, D2), jnp.float32),
        pltpu.VMEM((CH, D2), jnp.float32),
        pltpu.VMEM((ZR, D2), jnp.float32),
        pltpu.SemaphoreType.DMA,
        pltpu.SemaphoreType.DMA,
    ]
    if with_deg:
        out_type.append(jax.ShapeDtypeStruct((NPAD, DEGW), jnp.float32))
        scratch.insert(1, pltpu.VMEM_SHARED((NPAD, DEGW), jnp.float32))
        scratch.append(pltpu.VMEM((CH, DEGW), jnp.float32))
        scratch.append(pltpu.VMEM((ZR, DEGW), jnp.float32))

    def edge_phase(h_half, src3_hbm, dst3_hbm, sid, src_big, dst_big,
                   rows_a, rows_b, sem_a, sem_b, agg_sh,
                   deg_sh=None, ones_v=None):
        pltpu.sync_copy(src3_hbm.at[sid], src_big)
        pltpu.sync_copy(dst3_hbm.at[sid], dst_big)
        pltpu.async_copy(h_half.at[src_big.at[0]], rows_a, sem_a)

        def pair(i, carry):
            c = 2 * i
            pltpu.async_copy(h_half.at[src_big.at[c + 1]], rows_b, sem_b)
            pltpu.make_async_copy(h_half.at[src_big.at[0]], rows_a,
                                  sem_a).wait()
            pltpu.sync_copy(rows_a, agg_sh.at[dst_big.at[c]], add=True)
            if deg_sh is not None:
                pltpu.sync_copy(ones_v, deg_sh.at[dst_big.at[c]], add=True)

            @pl.when(c + 2 < NCHT)
            def _():
                pltpu.async_copy(h_half.at[src_big.at[c + 2]], rows_a, sem_a)

            pltpu.make_async_copy(h_half.at[src_big.at[0]], rows_b,
                                  sem_b).wait()
            pltpu.sync_copy(rows_b, agg_sh.at[dst_big.at[c + 1]], add=True)
            if deg_sh is not None:
                pltpu.sync_copy(ones_v, deg_sh.at[dst_big.at[c + 1]],
                                add=True)
            return carry

        lax.fori_loop(0, NCHT // 2, pair, 0)

    def body_deg(hlo_hbm, hhi_hbm, src3_hbm, dst3_hbm, z64_hbm, z16_hbm,
                 ones_hbm,
                 agg_out, deg_out,
                 agg_sh, deg_sh, src_big, dst_big, rows_a, rows_b, stage_v,
                 sem_a, sem_b, ones_v, dstage_v):
        cid = lax.axis_index("c")
        sid = lax.axis_index("s")
        row0 = sid * RPS
        pltpu.sync_copy(z64_hbm, stage_v)
        pltpu.sync_copy(z16_hbm, dstage_v)
        pltpu.sync_copy(ones_hbm, ones_v)

        def zstep(k, carry):
            r = row0 + k * ZR
            pltpu.sync_copy(stage_v, agg_sh.at[pl.ds(r, ZR)])
            pltpu.sync_copy(dstage_v, deg_sh.at[pl.ds(r, ZR)])
            return carry

        lax.fori_loop(0, NZ, zstep, 0)
        plsc.subcore_barrier()

        @pl.when(cid == 0)
        def _():
            edge_phase(hlo_hbm, src3_hbm, dst3_hbm, sid, src_big, dst_big,
                       rows_a, rows_b, sem_a, sem_b, agg_sh,
                       deg_sh=deg_sh, ones_v=ones_v)

        @pl.when(cid == 1)
        def _():
            edge_phase(hhi_hbm, src3_hbm, dst3_hbm, sid, src_big, dst_big,
                       rows_a, rows_b, sem_a, sem_b, agg_sh)

        plsc.subcore_barrier()

        def wstep(k, carry):
            r = row0 + k * ZR
            pltpu.sync_copy(agg_sh.at[pl.ds(r, ZR)], stage_v)
            pltpu.sync_copy(stage_v, agg_out.at[cid, pl.ds(r, ZR)])
            return carry

        lax.fori_loop(0, NZ, wstep, 0)

        @pl.when(cid == 0)
        def _():
            def dwstep(k, carry):
                r = row0 + k * ZR
                pltpu.sync_copy(deg_sh.at[pl.ds(r, ZR)], dstage_v)
                pltpu.sync_copy(dstage_v, deg_out.at[pl.ds(r, ZR)])
                return carry
            lax.fori_loop(0, NZ, dwstep, 0)

    def body_nodeg(hlo_hbm, hhi_hbm, src3_hbm, dst3_hbm, z64_hbm,
                   agg_out,
                   agg_sh, src_big, dst_big, rows_a, rows_b, stage_v,
                   sem_a, sem_b):
        cid = lax.axis_index("c")
        sid = lax.axis_index("s")
        row0 = sid * RPS
        pltpu.sync_copy(z64_hbm, stage_v)

        def zstep(k, carry):
            pltpu.sync_copy(stage_v, agg_sh.at[pl.ds(row0 + k * ZR, ZR)])
            return carry

        lax.fori_loop(0, NZ, zstep, 0)
        plsc.subcore_barrier()

        @pl.when(cid == 0)
        def _():
            edge_phase(hlo_hbm, src3_hbm, dst3_hbm, sid, src_big, dst_big,
                       rows_a, rows_b, sem_a, sem_b, agg_sh)

        @pl.when(cid == 1)
        def _():
            edge_phase(hhi_hbm, src3_hbm, dst3_hbm, sid, src_big, dst_big,
                       rows_a, rows_b, sem_a, sem_b, agg_sh)

        plsc.subcore_barrier()

        def wstep(k, carry):
            r = row0 + k * ZR
            pltpu.sync_copy(agg_sh.at[pl.ds(r, ZR)], stage_v)
            pltpu.sync_copy(stage_v, agg_out.at[cid, pl.ds(r, ZR)])
            return carry

        lax.fori_loop(0, NZ, wstep, 0)

    body = body_deg if with_deg else body_nodeg
    return pl.kernel(
        body, out_type=tuple(out_type), mesh=_MESH,
        scratch_types=tuple(scratch),
        compiler_params=pltpu.CompilerParams(use_tc_tiling_on_sc=False))


_sc_layer_deg = _make_sc_layer(True)
_sc_layer = _make_sc_layer(False)


BR = 1000
G = N // BR


def _enc_body(x_ref, w_ref, b_ref, olo_ref, ohi_ref):
    z = (jnp.dot(x_ref[...], w_ref[...],
                 preferred_element_type=jnp.float32) + b_ref[...])
    olo_ref[...] = z[:, :D2]
    ohi_ref[...] = z[:, D2:]


def _layer_body(agg_ref, deg_ref, hlo_ref, hhi_ref, wn_ref, wr_ref, b_ref,
                olo_ref, ohi_ref):
    agg = jnp.concatenate([agg_ref[0], agg_ref[1]], axis=1)
    h = jnp.concatenate([hlo_ref[...], hhi_ref[...]], axis=1)
    inv = 1.0 / jnp.maximum(deg_ref[:, 0:1], 1.0)
    z = (jnp.dot(agg * inv, wn_ref[...], preferred_element_type=jnp.float32)
         + jnp.dot(h, wr_ref[...], preferred_element_type=jnp.float32)
         + b_ref[...])
    z = jnp.maximum(z, 0.0)
    olo_ref[...] = z[:, :D2]
    ohi_ref[...] = z[:, D2:]


def _layer_pred_body(agg_ref, deg_ref, hlo_ref, hhi_ref, wn_ref, wr_ref,
                     b_ref, wp_ref, bp_ref, o_ref):
    agg = jnp.concatenate([agg_ref[0], agg_ref[1]], axis=1)
    h = jnp.concatenate([hlo_ref[...], hhi_ref[...]], axis=1)
    inv = 1.0 / jnp.maximum(deg_ref[:, 0:1], 1.0)
    z = (jnp.dot(agg * inv, wn_ref[...], preferred_element_type=jnp.float32)
         + jnp.dot(h, wr_ref[...], preferred_element_type=jnp.float32)
         + b_ref[...])
    z = jnp.maximum(z, 0.0)
    o_ref[...] = (jnp.dot(z, wp_ref[...], preferred_element_type=jnp.float32)
                  + bp_ref[...])


_row_spec = pl.BlockSpec((BR, D), lambda i: (i, 0))
_half_spec = pl.BlockSpec((BR, D2), lambda i: (i, 0))
_w_spec = pl.BlockSpec((D, D), lambda i: (0, 0))
_b_spec = pl.BlockSpec((1, D), lambda i: (0, 0))
_agg_spec = pl.BlockSpec((NC, BR, D2), lambda i: (0, i, 0))
_deg_spec = pl.BlockSpec((BR, DEGW), lambda i: (i, 0))
_half_sds = jax.ShapeDtypeStruct((N, D2), jnp.float32)

_encoder = pl.pallas_call(
    _enc_body, grid=(G,),
    in_specs=[_row_spec, _w_spec, _b_spec],
    out_specs=[_half_spec, _half_spec], out_shape=[_half_sds, _half_sds])

_tc_layer = pl.pallas_call(
    _layer_body, grid=(G,),
    in_specs=[_agg_spec, _deg_spec, _half_spec, _half_spec, _w_spec, _w_spec,
              _b_spec],
    out_specs=[_half_spec, _half_spec], out_shape=[_half_sds, _half_sds])

_tc_layer_pred = pl.pallas_call(
    _layer_pred_body, grid=(G,),
    in_specs=[_agg_spec, _deg_spec, _half_spec, _half_spec, _w_spec, _w_spec,
              _b_spec, _w_spec, _b_spec],
    out_specs=_row_spec,
    out_shape=jax.ShapeDtypeStruct((N, D), jnp.float32))


def kernel(nodes, edge_index, W_enc, b_enc, W_root, W_neigh, b_conv,
           W_pred, b_pred):
    src = edge_index[0].reshape(NS, NCHT, CH)
    dst = edge_index[1].reshape(NS, NCHT, CH)
    z64 = jnp.zeros((ZR, D2), jnp.float32)
    z16 = jnp.zeros((ZR, DEGW), jnp.float32)
    ones_ch = jnp.ones((CH, DEGW), jnp.float32)

    h0lo, h0hi = _encoder(nodes, W_enc, b_enc.reshape(1, D))
    agg1, deg = _sc_layer_deg(h0lo, h0hi, src, dst, z64, z16, ones_ch)
    h1lo, h1hi = _tc_layer(agg1, deg, h0lo, h0hi, W_neigh[0], W_root[0],
                           b_conv[0].reshape(1, D))
    (agg2,) = _sc_layer(h1lo, h1hi, src, dst, z64)
    return _tc_layer_pred(agg2, deg, h1lo, h1hi, W_neigh[1], W_root[1],
                          b_conv[1].reshape(1, D), W_pred,
                          b_pred.reshape(1, D))


# speedup vs baseline: 10.1757x; 1.1279x over previous
import jax
import jax.numpy as jnp
from jax import lax
from jax.experimental import pallas as pl
from jax.experimental.pallas import tpu as pltpu
from jax.experimental.pallas import tpu_sc as plsc

N = 10000
E = 320000
D = 128
D2 = D // 2

NC = 2
NS = 16
CH = 128
NCHT = 158
EPAD = NS * NCHT * CH
PADE = EPAD - E
NPAD = 10240
RPS = NPAD // NS
DEGW = 16
ZR = 160
NZ = RPS // ZR

_MESH = plsc.VectorSubcoreMesh(core_axis_name="c", subcore_axis_name="s")


def _make_sc_layer(with_deg):
    out_type = [jax.ShapeDtypeStruct((NC, NPAD, D2), jnp.float32)]
    scratch = [
        pltpu.VMEM_SHARED((NPAD, D2), jnp.float32),
        pltpu.VMEM((NCHT, CH), jnp.int32),
        pltpu.VMEM((NCHT, CH), jnp.int32),
        pltpu.VMEM((CH, D2), jnp.float32),
        pltpu.VMEM((CH, D2), jnp.float32),
        pltpu.VMEM((ZR, D2), jnp.float32),
        pltpu.SemaphoreType.DMA,
        pltpu.SemaphoreType.DMA,
    ]
    if with_deg:
        out_type.append(jax.ShapeDtypeStruct((NC, NPAD, DEGW), jnp.float32))
        scratch.insert(1, pltpu.VMEM_SHARED((NPAD, DEGW), jnp.float32))
        scratch.append(pltpu.VMEM((CH, DEGW), jnp.float32))
        scratch.append(pltpu.VMEM((ZR, DEGW), jnp.float32))

    def edge_phase(h_half, src3_hbm, dst3_hbm, sid, src_big, dst_big,
                   rows_a, rows_b, sem_a, sem_b, agg_sh,
                   deg_parity=None, deg_sh=None, ones_v=None):
        pltpu.sync_copy(src3_hbm.at[sid], src_big)
        pltpu.sync_copy(dst3_hbm.at[sid], dst_big)
        pltpu.async_copy(h_half.at[src_big.at[0]], rows_a, sem_a)

        def pair(i, carry):
            c = 2 * i
            pltpu.async_copy(h_half.at[src_big.at[c + 1]], rows_b, sem_b)
            pltpu.make_async_copy(h_half.at[src_big.at[0]], rows_a,
                                  sem_a).wait()
            pltpu.sync_copy(rows_a, agg_sh.at[dst_big.at[c]], add=True)
            if deg_parity == 0:
                pltpu.sync_copy(ones_v, deg_sh.at[dst_big.at[c]], add=True)

            @pl.when(c + 2 < NCHT)
            def _():
                pltpu.async_copy(h_half.at[src_big.at[c + 2]], rows_a, sem_a)

            pltpu.make_async_copy(h_half.at[src_big.at[0]], rows_b,
                                  sem_b).wait()
            pltpu.sync_copy(rows_b, agg_sh.at[dst_big.at[c + 1]], add=True)
            if deg_parity == 1:
                pltpu.sync_copy(ones_v, deg_sh.at[dst_big.at[c + 1]],
                                add=True)
            return carry

        lax.fori_loop(0, NCHT // 2, pair, 0)

    def body_deg(hlo_hbm, hhi_hbm, src3_hbm, dst3_hbm, z64_hbm, z16_hbm,
                 ones_hbm,
                 agg_out, deg_out,
                 agg_sh, deg_sh, src_big, dst_big, rows_a, rows_b, stage_v,
                 sem_a, sem_b, ones_v, dstage_v):
        cid = lax.axis_index("c")
        sid = lax.axis_index("s")
        row0 = sid * RPS
        pltpu.sync_copy(z64_hbm, stage_v)
        pltpu.sync_copy(z16_hbm, dstage_v)
        pltpu.sync_copy(ones_hbm, ones_v)

        def zstep(k, carry):
            r = row0 + k * ZR
            pltpu.sync_copy(stage_v, agg_sh.at[pl.ds(r, ZR)])
            pltpu.sync_copy(dstage_v, deg_sh.at[pl.ds(r, ZR)])
            return carry

        lax.fori_loop(0, NZ, zstep, 0)
        plsc.subcore_barrier()

        @pl.when(cid == 0)
        def _():
            edge_phase(hlo_hbm, src3_hbm, dst3_hbm, sid, src_big, dst_big,
                       rows_a, rows_b, sem_a, sem_b, agg_sh,
                       deg_parity=0, deg_sh=deg_sh, ones_v=ones_v)

        @pl.when(cid == 1)
        def _():
            edge_phase(hhi_hbm, src3_hbm, dst3_hbm, sid, src_big, dst_big,
                       rows_a, rows_b, sem_a, sem_b, agg_sh,
                       deg_parity=1, deg_sh=deg_sh, ones_v=ones_v)

        plsc.subcore_barrier()

        def wstep(k, carry):
            r = row0 + k * ZR
            pltpu.sync_copy(agg_sh.at[pl.ds(r, ZR)], stage_v)
            pltpu.sync_copy(stage_v, agg_out.at[cid, pl.ds(r, ZR)])
            return carry

        lax.fori_loop(0, NZ, wstep, 0)

        def dwstep(k, carry):
            r = row0 + k * ZR
            pltpu.sync_copy(deg_sh.at[pl.ds(r, ZR)], dstage_v)
            pltpu.sync_copy(dstage_v, deg_out.at[cid, pl.ds(r, ZR)])
            return carry

        lax.fori_loop(0, NZ, dwstep, 0)

    def body_nodeg(hlo_hbm, hhi_hbm, src3_hbm, dst3_hbm, z64_hbm,
                   agg_out,
                   agg_sh, src_big, dst_big, rows_a, rows_b, stage_v,
                   sem_a, sem_b):
        cid = lax.axis_index("c")
        sid = lax.axis_index("s")
        row0 = sid * RPS
        pltpu.sync_copy(z64_hbm, stage_v)

        def zstep(k, carry):
            pltpu.sync_copy(stage_v, agg_sh.at[pl.ds(row0 + k * ZR, ZR)])
            return carry

        lax.fori_loop(0, NZ, zstep, 0)
        plsc.subcore_barrier()

        @pl.when(cid == 0)
        def _():
            edge_phase(hlo_hbm, src3_hbm, dst3_hbm, sid, src_big, dst_big,
                       rows_a, rows_b, sem_a, sem_b, agg_sh)

        @pl.when(cid == 1)
        def _():
            edge_phase(hhi_hbm, src3_hbm, dst3_hbm, sid, src_big, dst_big,
                       rows_a, rows_b, sem_a, sem_b, agg_sh)

        plsc.subcore_barrier()

        def wstep(k, carry):
            r = row0 + k * ZR
            pltpu.sync_copy(agg_sh.at[pl.ds(r, ZR)], stage_v)
            pltpu.sync_copy(stage_v, agg_out.at[cid, pl.ds(r, ZR)])
            return carry

        lax.fori_loop(0, NZ, wstep, 0)

    body = body_deg if with_deg else body_nodeg
    return pl.kernel(
        body, out_type=tuple(out_type), mesh=_MESH,
        scratch_types=tuple(scratch),
        compiler_params=pltpu.CompilerParams(use_tc_tiling_on_sc=False))


_sc_layer_deg = _make_sc_layer(True)
_sc_layer = _make_sc_layer(False)


BR = 1000
G = N // BR


def _enc_body(x_ref, w_ref, b_ref, olo_ref, ohi_ref):
    z = (jnp.dot(x_ref[...], w_ref[...],
                 preferred_element_type=jnp.float32) + b_ref[...])
    olo_ref[...] = z[:, :D2]
    ohi_ref[...] = z[:, D2:]


def _layer_body(agg_ref, deg_ref, hlo_ref, hhi_ref, wn_ref, wr_ref, b_ref,
                olo_ref, ohi_ref):
    agg = jnp.concatenate([agg_ref[0], agg_ref[1]], axis=1)
    h = jnp.concatenate([hlo_ref[...], hhi_ref[...]], axis=1)
    inv = 1.0 / jnp.maximum(deg_ref[0, :, 0:1] + deg_ref[1, :, 0:1], 1.0)
    z = (jnp.dot(agg * inv, wn_ref[...], preferred_element_type=jnp.float32)
         + jnp.dot(h, wr_ref[...], preferred_element_type=jnp.float32)
         + b_ref[...])
    z = jnp.maximum(z, 0.0)
    olo_ref[...] = z[:, :D2]
    ohi_ref[...] = z[:, D2:]


def _layer_pred_body(agg_ref, deg_ref, hlo_ref, hhi_ref, wn_ref, wr_ref,
                     b_ref, wp_ref, bp_ref, o_ref):
    agg = jnp.concatenate([agg_ref[0], agg_ref[1]], axis=1)
    h = jnp.concatenate([hlo_ref[...], hhi_ref[...]], axis=1)
    inv = 1.0 / jnp.maximum(deg_ref[0, :, 0:1] + deg_ref[1, :, 0:1], 1.0)
    z = (jnp.dot(agg * inv, wn_ref[...], preferred_element_type=jnp.float32)
         + jnp.dot(h, wr_ref[...], preferred_element_type=jnp.float32)
         + b_ref[...])
    z = jnp.maximum(z, 0.0)
    o_ref[...] = (jnp.dot(z, wp_ref[...], preferred_element_type=jnp.float32)
                  + bp_ref[...])


_row_spec = pl.BlockSpec((BR, D), lambda i: (i, 0))
_half_spec = pl.BlockSpec((BR, D2), lambda i: (i, 0))
_w_spec = pl.BlockSpec((D, D), lambda i: (0, 0))
_b_spec = pl.BlockSpec((1, D), lambda i: (0, 0))
_agg_spec = pl.BlockSpec((NC, BR, D2), lambda i: (0, i, 0))
_deg_spec = pl.BlockSpec((NC, BR, DEGW), lambda i: (0, i, 0))
_half_sds = jax.ShapeDtypeStruct((N, D2), jnp.float32)

_encoder = pl.pallas_call(
    _enc_body, grid=(G,),
    in_specs=[_row_spec, _w_spec, _b_spec],
    out_specs=[_half_spec, _half_spec], out_shape=[_half_sds, _half_sds])

_tc_layer = pl.pallas_call(
    _layer_body, grid=(G,),
    in_specs=[_agg_spec, _deg_spec, _half_spec, _half_spec, _w_spec, _w_spec,
              _b_spec],
    out_specs=[_half_spec, _half_spec], out_shape=[_half_sds, _half_sds])

_tc_layer_pred = pl.pallas_call(
    _layer_pred_body, grid=(G,),
    in_specs=[_agg_spec, _deg_spec, _half_spec, _half_spec, _w_spec, _w_spec,
              _b_spec, _w_spec, _b_spec],
    out_specs=_row_spec,
    out_shape=jax.ShapeDtypeStruct((N, D), jnp.float32))


def kernel(nodes, edge_index, W_enc, b_enc, W_root, W_neigh, b_conv,
           W_pred, b_pred):
    pad_i = jnp.arange(PADE, dtype=jnp.int32)
    pad_src = (pad_i * 131) % N
    pad_dst = N + pad_i % (NPAD - N)
    src = jnp.concatenate([edge_index[0], pad_src]).reshape(NS, NCHT, CH)
    dst = jnp.concatenate([edge_index[1], pad_dst]).reshape(NS, NCHT, CH)
    z64 = jnp.zeros((ZR, D2), jnp.float32)
    z16 = jnp.zeros((ZR, DEGW), jnp.float32)
    ones_ch = jnp.ones((CH, DEGW), jnp.float32)

    h0lo, h0hi = _encoder(nodes, W_enc, b_enc.reshape(1, D))
    agg1, deg = _sc_layer_deg(h0lo, h0hi, src, dst, z64, z16, ones_ch)
    h1lo, h1hi = _tc_layer(agg1, deg, h0lo, h0hi, W_neigh[0], W_root[0],
                           b_conv[0].reshape(1, D))
    (agg2,) = _sc_layer(h1lo, h1hi, src, dst, z64)
    return _tc_layer_pred(agg2, deg, h1lo, h1hi, W_neigh[1], W_root[1],
                          b_conv[1].reshape(1, D), W_pred,
                          b_pred.reshape(1, D))


# speedup vs baseline: 12.0987x; 1.1890x over previous
import jax
import jax.numpy as jnp
from jax import lax
from jax.experimental import pallas as pl
from jax.experimental.pallas import tpu as pltpu
from jax.experimental.pallas import tpu_sc as plsc

N = 10000
E = 320000
D = 128
D2 = D // 2

NC = 2
NS = 16
CH = 128
NCHT = 159
EPAD = NS * NCHT * CH
PADE = EPAD - E
NPAD = 10240
RPS = NPAD // NS
DEGW = 8
ZR = 80
NZ = RPS // ZR

_MESH = plsc.VectorSubcoreMesh(core_axis_name="c", subcore_axis_name="s")


def _make_sc_layer(with_deg):
    out_type = [jax.ShapeDtypeStruct((NC, NPAD, D2), jnp.float32)]
    scratch = [
        pltpu.VMEM_SHARED((NPAD, D2), jnp.float32),
        pltpu.VMEM((NCHT, CH), jnp.int32),
        pltpu.VMEM((NCHT, CH), jnp.int32),
        pltpu.VMEM((CH, D2), jnp.float32),
        pltpu.VMEM((CH, D2), jnp.float32),
        pltpu.VMEM((CH, D2), jnp.float32),
        pltpu.VMEM((ZR, D2), jnp.float32),
        pltpu.SemaphoreType.DMA,
        pltpu.SemaphoreType.DMA,
        pltpu.SemaphoreType.DMA,
    ]
    if with_deg:
        out_type.append(jax.ShapeDtypeStruct((NC, NPAD, DEGW), jnp.float32))
        scratch.insert(1, pltpu.VMEM_SHARED((NPAD, DEGW), jnp.float32))
        scratch.append(pltpu.VMEM((CH, DEGW), jnp.float32))
        scratch.append(pltpu.VMEM((ZR, DEGW), jnp.float32))

    def edge_phase(h_half, src3_hbm, dst3_hbm, sid, src_big, dst_big,
                   rows, gsems, agg_sh,
                   deg_lo=None, deg_hi=None, deg_sh=None, ones_v=None):
        pltpu.sync_copy(src3_hbm.at[sid], src_big)
        pltpu.sync_copy(dst3_hbm.at[sid], dst_big)
        for j in range(3):
            pltpu.async_copy(h_half.at[src_big.at[j]], rows[j], gsems[j])

        def ring(i, carry):
            c = 3 * i
            for j in range(3):
                pltpu.make_async_copy(h_half.at[src_big.at[0]], rows[j],
                                      gsems[j]).wait()
                pltpu.sync_copy(rows[j], agg_sh.at[dst_big.at[c + j]],
                                add=True)
                if deg_sh is not None:
                    @pl.when(jnp.logical_and(c + j >= deg_lo,
                                             c + j < deg_hi))
                    def _(j=j, c=c):
                        pltpu.sync_copy(ones_v, deg_sh.at[dst_big.at[c + j]],
                                        add=True)

                @pl.when(c + 3 + j < NCHT)
                def _(j=j, c=c):
                    pltpu.async_copy(h_half.at[src_big.at[c + 3 + j]],
                                     rows[j], gsems[j])
            return carry

        lax.fori_loop(0, NCHT // 3, ring, 0)

    def body_deg(hlo_hbm, hhi_hbm, src3_hbm, dst3_hbm, z64_hbm, z16_hbm,
                 ones_hbm,
                 agg_out, deg_out,
                 agg_sh, deg_sh, src_big, dst_big, r0, r1, r2, stage_v,
                 g0, g1, g2, ones_v, dstage_v):
        rows = (r0, r1, r2)
        gsems = (g0, g1, g2)
        cid = lax.axis_index("c")
        sid = lax.axis_index("s")
        row0 = sid * RPS
        pltpu.sync_copy(z64_hbm, stage_v)
        pltpu.sync_copy(z16_hbm, dstage_v)
        pltpu.sync_copy(ones_hbm, ones_v)

        def zstep(k, carry):
            r = row0 + k * ZR
            pltpu.sync_copy(stage_v, agg_sh.at[pl.ds(r, ZR)])
            pltpu.sync_copy(dstage_v, deg_sh.at[pl.ds(r, ZR)])
            return carry

        lax.fori_loop(0, NZ, zstep, 0)
        plsc.subcore_barrier()

        @pl.when(cid == 0)
        def _():
            edge_phase(hlo_hbm, src3_hbm, dst3_hbm, sid, src_big, dst_big,
                       rows, gsems, agg_sh, deg_lo=0, deg_hi=80,
                       deg_sh=deg_sh, ones_v=ones_v)

        @pl.when(cid == 1)
        def _():
            edge_phase(hhi_hbm, src3_hbm, dst3_hbm, sid, src_big, dst_big,
                       rows, gsems, agg_sh, deg_lo=80, deg_hi=NCHT,
                       deg_sh=deg_sh, ones_v=ones_v)

        plsc.subcore_barrier()

        def wstep(k, carry):
            r = row0 + k * ZR
            pltpu.sync_copy(agg_sh.at[pl.ds(r, ZR)], stage_v)
            pltpu.sync_copy(stage_v, agg_out.at[cid, pl.ds(r, ZR)])
            return carry

        lax.fori_loop(0, NZ, wstep, 0)

        def dwstep(k, carry):
            r = row0 + k * ZR
            pltpu.sync_copy(deg_sh.at[pl.ds(r, ZR)], dstage_v)
            pltpu.sync_copy(dstage_v, deg_out.at[cid, pl.ds(r, ZR)])
            return carry

        lax.fori_loop(0, NZ, dwstep, 0)

    def body_nodeg(hlo_hbm, hhi_hbm, src3_hbm, dst3_hbm, z64_hbm,
                   agg_out,
                   agg_sh, src_big, dst_big, r0, r1, r2, stage_v,
                   g0, g1, g2):
        rows = (r0, r1, r2)
        gsems = (g0, g1, g2)
        cid = lax.axis_index("c")
        sid = lax.axis_index("s")
        row0 = sid * RPS
        pltpu.sync_copy(z64_hbm, stage_v)

        def zstep(k, carry):
            pltpu.sync_copy(stage_v, agg_sh.at[pl.ds(row0 + k * ZR, ZR)])
            return carry

        lax.fori_loop(0, NZ, zstep, 0)
        plsc.subcore_barrier()

        @pl.when(cid == 0)
        def _():
            edge_phase(hlo_hbm, src3_hbm, dst3_hbm, sid, src_big, dst_big,
                       rows, gsems, agg_sh)

        @pl.when(cid == 1)
        def _():
            edge_phase(hhi_hbm, src3_hbm, dst3_hbm, sid, src_big, dst_big,
                       rows, gsems, agg_sh)

        plsc.subcore_barrier()

        def wstep(k, carry):
            r = row0 + k * ZR
            pltpu.sync_copy(agg_sh.at[pl.ds(r, ZR)], stage_v)
            pltpu.sync_copy(stage_v, agg_out.at[cid, pl.ds(r, ZR)])
            return carry

        lax.fori_loop(0, NZ, wstep, 0)

    body = body_deg if with_deg else body_nodeg
    return pl.kernel(
        body, out_type=tuple(out_type), mesh=_MESH,
        scratch_types=tuple(scratch),
        compiler_params=pltpu.CompilerParams(use_tc_tiling_on_sc=False))


_sc_layer_deg = _make_sc_layer(True)
_sc_layer = _make_sc_layer(False)


BR = 1000
G = N // BR


def _enc_body(x_ref, w_ref, b_ref, olo_ref, ohi_ref):
    z = (jnp.dot(x_ref[...], w_ref[...],
                 preferred_element_type=jnp.float32) + b_ref[...])
    olo_ref[...] = z[:, :D2]
    ohi_ref[...] = z[:, D2:]


def _layer_body(agg_ref, deg_ref, hlo_ref, hhi_ref, wn_ref, wr_ref, b_ref,
                olo_ref, ohi_ref):
    agg = jnp.concatenate([agg_ref[0], agg_ref[1]], axis=1)
    h = jnp.concatenate([hlo_ref[...], hhi_ref[...]], axis=1)
    inv = 1.0 / jnp.maximum(deg_ref[0, :, 0:1] + deg_ref[1, :, 0:1], 1.0)
    z = (jnp.dot(agg * inv, wn_ref[...], preferred_element_type=jnp.float32)
         + jnp.dot(h, wr_ref[...], preferred_element_type=jnp.float32)
         + b_ref[...])
    z = jnp.maximum(z, 0.0)
    olo_ref[...] = z[:, :D2]
    ohi_ref[...] = z[:, D2:]


def _layer_pred_body(agg_ref, deg_ref, hlo_ref, hhi_ref, wn_ref, wr_ref,
                     b_ref, wp_ref, bp_ref, o_ref):
    agg = jnp.concatenate([agg_ref[0], agg_ref[1]], axis=1)
    h = jnp.concatenate([hlo_ref[...], hhi_ref[...]], axis=1)
    inv = 1.0 / jnp.maximum(deg_ref[0, :, 0:1] + deg_ref[1, :, 0:1], 1.0)
    z = (jnp.dot(agg * inv, wn_ref[...], preferred_element_type=jnp.float32)
         + jnp.dot(h, wr_ref[...], preferred_element_type=jnp.float32)
         + b_ref[...])
    z = jnp.maximum(z, 0.0)
    o_ref[...] = (jnp.dot(z, wp_ref[...], preferred_element_type=jnp.float32)
                  + bp_ref[...])


_row_spec = pl.BlockSpec((BR, D), lambda i: (i, 0))
_half_spec = pl.BlockSpec((BR, D2), lambda i: (i, 0))
_w_spec = pl.BlockSpec((D, D), lambda i: (0, 0))
_b_spec = pl.BlockSpec((1, D), lambda i: (0, 0))
_agg_spec = pl.BlockSpec((NC, BR, D2), lambda i: (0, i, 0))
_deg_spec = pl.BlockSpec((NC, BR, DEGW), lambda i: (0, i, 0))
_half_sds = jax.ShapeDtypeStruct((N, D2), jnp.float32)

_encoder = pl.pallas_call(
    _enc_body, grid=(G,),
    in_specs=[_row_spec, _w_spec, _b_spec],
    out_specs=[_half_spec, _half_spec], out_shape=[_half_sds, _half_sds])

_tc_layer = pl.pallas_call(
    _layer_body, grid=(G,),
    in_specs=[_agg_spec, _deg_spec, _half_spec, _half_spec, _w_spec, _w_spec,
              _b_spec],
    out_specs=[_half_spec, _half_spec], out_shape=[_half_sds, _half_sds])

_tc_layer_pred = pl.pallas_call(
    _layer_pred_body, grid=(G,),
    in_specs=[_agg_spec, _deg_spec, _half_spec, _half_spec, _w_spec, _w_spec,
              _b_spec, _w_spec, _b_spec],
    out_specs=_row_spec,
    out_shape=jax.ShapeDtypeStruct((N, D), jnp.float32))


def kernel(nodes, edge_index, W_enc, b_enc, W_root, W_neigh, b_conv,
           W_pred, b_pred):
    pad_i = jnp.arange(PADE, dtype=jnp.int32)
    pad_src = (pad_i * 131) % N
    pad_dst = N + pad_i % (NPAD - N)
    src = jnp.concatenate([edge_index[0], pad_src]).reshape(NS, NCHT, CH)
    dst = jnp.concatenate([edge_index[1], pad_dst]).reshape(NS, NCHT, CH)
    z64 = jnp.zeros((ZR, D2), jnp.float32)
    z16 = jnp.zeros((ZR, DEGW), jnp.float32)
    ones_ch = jnp.ones((CH, DEGW), jnp.float32)

    h0lo, h0hi = _encoder(nodes, W_enc, b_enc.reshape(1, D))
    agg1, deg = _sc_layer_deg(h0lo, h0hi, src, dst, z64, z16, ones_ch)
    h1lo, h1hi = _tc_layer(agg1, deg, h0lo, h0hi, W_neigh[0], W_root[0],
                           b_conv[0].reshape(1, D))
    (agg2,) = _sc_layer(h1lo, h1hi, src, dst, z64)
    return _tc_layer_pred(agg2, deg, h1lo, h1hi, W_neigh[1], W_root[1],
                          b_conv[1].reshape(1, D), W_pred,
                          b_pred.reshape(1, D))


# speedup vs baseline: 12.2499x; 1.0125x over previous
import jax
import jax.numpy as jnp
from jax import lax
from jax.experimental import pallas as pl
from jax.experimental.pallas import tpu as pltpu
from jax.experimental.pallas import tpu_sc as plsc

N = 10000
E = 320000
D = 128
D2 = D // 2

NC = 2
NS = 16
CH = 128
NCHT = 159
EPAD = NS * NCHT * CH
PADE = EPAD - E
NPAD = 10240
RPS = NPAD // NS
DEGW = 8
ZR = 80
NZ = RPS // ZR

_MESH = plsc.VectorSubcoreMesh(core_axis_name="c", subcore_axis_name="s")


def _make_sc_layer(with_deg):
    out_type = [jax.ShapeDtypeStruct((NC, NPAD, D2), jnp.float32)]
    scratch = [
        pltpu.VMEM_SHARED((NPAD, D2), jnp.float32),
        pltpu.VMEM((NCHT, CH), jnp.int32),
        pltpu.VMEM((NCHT, CH), jnp.int32),
        pltpu.VMEM((CH, D2), jnp.float32),
        pltpu.VMEM((CH, D2), jnp.float32),
        pltpu.VMEM((CH, D2), jnp.float32),
        pltpu.VMEM((ZR, D2), jnp.float32),
        pltpu.SemaphoreType.DMA,
        pltpu.SemaphoreType.DMA,
        pltpu.SemaphoreType.DMA,
        pltpu.SemaphoreType.DMA,
        pltpu.SemaphoreType.DMA,
    ]
    if with_deg:
        out_type.append(jax.ShapeDtypeStruct((NC, NPAD, DEGW), jnp.float32))
        scratch.insert(1, pltpu.VMEM_SHARED((NPAD, DEGW), jnp.float32))
        scratch.append(pltpu.VMEM((CH, DEGW), jnp.float32))
        scratch.append(pltpu.VMEM((ZR, DEGW), jnp.float32))

    def edge_phase(h_half, src3_hbm, dst3_hbm, sid, src_big, dst_big,
                   rows, gsems, agg_sh,
                   deg_lo=None, deg_hi=None, deg_sh=None, ones_v=None):
        pltpu.sync_copy(src3_hbm.at[sid], src_big)
        pltpu.sync_copy(dst3_hbm.at[sid], dst_big)
        for j in range(3):
            pltpu.async_copy(h_half.at[src_big.at[j]], rows[j], gsems[j])

        def ring(i, carry):
            c = 3 * i
            for j in range(3):
                pltpu.make_async_copy(h_half.at[src_big.at[0]], rows[j],
                                      gsems[j]).wait()
                pltpu.sync_copy(rows[j], agg_sh.at[dst_big.at[c + j]],
                                add=True)
                if deg_sh is not None:
                    @pl.when(jnp.logical_and(c + j >= deg_lo,
                                             c + j < deg_hi))
                    def _(j=j, c=c):
                        pltpu.sync_copy(ones_v, deg_sh.at[dst_big.at[c + j]],
                                        add=True)

                @pl.when(c + 3 + j < NCHT)
                def _(j=j, c=c):
                    pltpu.async_copy(h_half.at[src_big.at[c + 3 + j]],
                                     rows[j], gsems[j])
            return carry

        lax.fori_loop(0, NCHT // 3, ring, 0)

    def init_zero(stage_v, sh_ref, row0, sems):
        for k in range(NZ):
            pltpu.async_copy(stage_v, sh_ref.at[pl.ds(row0 + k * ZR, ZR)],
                             sems[k % 3])
        for k in range(NZ):
            pltpu.make_async_copy(stage_v, sh_ref.at[pl.ds(row0, ZR)],
                                  sems[k % 3]).wait()

    def pp_writeout(sh_ref, out_at, row0, rows, lsems, wsems):
        wb = (rows[0].at[pl.ds(0, ZR)], rows[1].at[pl.ds(0, ZR)])
        pltpu.async_copy(sh_ref.at[pl.ds(row0, ZR)], wb[0], lsems[0])
        for k in range(NZ):
            j = k % 2
            pltpu.make_async_copy(sh_ref.at[pl.ds(row0, ZR)], wb[j],
                                  lsems[j]).wait()
            pltpu.async_copy(wb[j], out_at(row0 + k * ZR), wsems[j])
            if k + 1 < NZ:
                jn = (k + 1) % 2
                if k >= 1:
                    pltpu.make_async_copy(wb[jn], out_at(row0),
                                          wsems[jn]).wait()
                pltpu.async_copy(sh_ref.at[pl.ds(row0 + (k + 1) * ZR, ZR)],
                                 wb[jn], lsems[jn])
        pltpu.make_async_copy(wb[(NZ - 2) % 2], out_at(row0),
                              wsems[(NZ - 2) % 2]).wait()
        pltpu.make_async_copy(wb[(NZ - 1) % 2], out_at(row0),
                              wsems[(NZ - 1) % 2]).wait()

    def body_deg(hlo_hbm, hhi_hbm, src3_hbm, dst3_hbm, z64_hbm, z16_hbm,
                 ones_hbm,
                 agg_out, deg_out,
                 agg_sh, deg_sh, src_big, dst_big, r0, r1, r2, stage_v,
                 g0, g1, g2, w0, w1, ones_v, dstage_v):
        rows = (r0, r1, r2)
        gsems = (g0, g1, g2)
        wsems = (w0, w1)
        cid = lax.axis_index("c")
        sid = lax.axis_index("s")
        row0 = sid * RPS
        pltpu.sync_copy(z64_hbm, stage_v)
        pltpu.sync_copy(z16_hbm, dstage_v)
        pltpu.sync_copy(ones_hbm, ones_v)
        init_zero(stage_v, agg_sh, row0, gsems)
        init_zero(dstage_v, deg_sh, row0, gsems)
        plsc.subcore_barrier()

        @pl.when(cid == 0)
        def _():
            edge_phase(hlo_hbm, src3_hbm, dst3_hbm, sid, src_big, dst_big,
                       rows, gsems, agg_sh, deg_lo=0, deg_hi=80,
                       deg_sh=deg_sh, ones_v=ones_v)

        @pl.when(cid == 1)
        def _():
            edge_phase(hhi_hbm, src3_hbm, dst3_hbm, sid, src_big, dst_big,
                       rows, gsems, agg_sh, deg_lo=80, deg_hi=NCHT,
                       deg_sh=deg_sh, ones_v=ones_v)

        plsc.subcore_barrier()

        pp_writeout(agg_sh, lambda r: agg_out.at[cid, pl.ds(r, ZR)],
                    row0, rows, gsems, wsems)

        def dwstep(k, carry):
            r = row0 + k * ZR
            pltpu.sync_copy(deg_sh.at[pl.ds(r, ZR)], dstage_v)
            pltpu.sync_copy(dstage_v, deg_out.at[cid, pl.ds(r, ZR)])
            return carry

        lax.fori_loop(0, NZ, dwstep, 0)

    def body_nodeg(hlo_hbm, hhi_hbm, src3_hbm, dst3_hbm, z64_hbm,
                   agg_out,
                   agg_sh, src_big, dst_big, r0, r1, r2, stage_v,
                   g0, g1, g2, w0, w1):
        rows = (r0, r1, r2)
        gsems = (g0, g1, g2)
        wsems = (w0, w1)
        cid = lax.axis_index("c")
        sid = lax.axis_index("s")
        row0 = sid * RPS
        pltpu.sync_copy(z64_hbm, stage_v)
        init_zero(stage_v, agg_sh, row0, gsems)
        plsc.subcore_barrier()

        @pl.when(cid == 0)
        def _():
            edge_phase(hlo_hbm, src3_hbm, dst3_hbm, sid, src_big, dst_big,
                       rows, gsems, agg_sh)

        @pl.when(cid == 1)
        def _():
            edge_phase(hhi_hbm, src3_hbm, dst3_hbm, sid, src_big, dst_big,
                       rows, gsems, agg_sh)

        plsc.subcore_barrier()

        pp_writeout(agg_sh, lambda r: agg_out.at[cid, pl.ds(r, ZR)],
                    row0, rows, gsems, wsems)

    body = body_deg if with_deg else body_nodeg
    return pl.kernel(
        body, out_type=tuple(out_type), mesh=_MESH,
        scratch_types=tuple(scratch),
        compiler_params=pltpu.CompilerParams(use_tc_tiling_on_sc=False))


_sc_layer_deg = _make_sc_layer(True)
_sc_layer = _make_sc_layer(False)


BR = 1000
G = N // BR


def _enc_body(x_ref, w_ref, b_ref, olo_ref, ohi_ref):
    z = (jnp.dot(x_ref[...], w_ref[...],
                 preferred_element_type=jnp.float32) + b_ref[...])
    olo_ref[...] = z[:, :D2]
    ohi_ref[...] = z[:, D2:]


def _layer_body(agg_ref, deg_ref, hlo_ref, hhi_ref, wn_ref, wr_ref, b_ref,
                olo_ref, ohi_ref):
    agg = jnp.concatenate([agg_ref[0], agg_ref[1]], axis=1)
    h = jnp.concatenate([hlo_ref[...], hhi_ref[...]], axis=1)
    inv = 1.0 / jnp.maximum(deg_ref[0, :, 0:1] + deg_ref[1, :, 0:1], 1.0)
    z = (jnp.dot(agg * inv, wn_ref[...], preferred_element_type=jnp.float32)
         + jnp.dot(h, wr_ref[...], preferred_element_type=jnp.float32)
         + b_ref[...])
    z = jnp.maximum(z, 0.0)
    olo_ref[...] = z[:, :D2]
    ohi_ref[...] = z[:, D2:]


def _layer_pred_body(agg_ref, deg_ref, hlo_ref, hhi_ref, wn_ref, wr_ref,
                     b_ref, wp_ref, bp_ref, o_ref):
    agg = jnp.concatenate([agg_ref[0], agg_ref[1]], axis=1)
    h = jnp.concatenate([hlo_ref[...], hhi_ref[...]], axis=1)
    inv = 1.0 / jnp.maximum(deg_ref[0, :, 0:1] + deg_ref[1, :, 0:1], 1.0)
    z = (jnp.dot(agg * inv, wn_ref[...], preferred_element_type=jnp.float32)
         + jnp.dot(h, wr_ref[...], preferred_element_type=jnp.float32)
         + b_ref[...])
    z = jnp.maximum(z, 0.0)
    o_ref[...] = (jnp.dot(z, wp_ref[...], preferred_element_type=jnp.float32)
                  + bp_ref[...])


_row_spec = pl.BlockSpec((BR, D), lambda i: (i, 0))
_half_spec = pl.BlockSpec((BR, D2), lambda i: (i, 0))
_w_spec = pl.BlockSpec((D, D), lambda i: (0, 0))
_b_spec = pl.BlockSpec((1, D), lambda i: (0, 0))
_agg_spec = pl.BlockSpec((NC, BR, D2), lambda i: (0, i, 0))
_deg_spec = pl.BlockSpec((NC, BR, DEGW), lambda i: (0, i, 0))
_half_sds = jax.ShapeDtypeStruct((N, D2), jnp.float32)

_encoder = pl.pallas_call(
    _enc_body, grid=(G,),
    in_specs=[_row_spec, _w_spec, _b_spec],
    out_specs=[_half_spec, _half_spec], out_shape=[_half_sds, _half_sds])

_tc_layer = pl.pallas_call(
    _layer_body, grid=(G,),
    in_specs=[_agg_spec, _deg_spec, _half_spec, _half_spec, _w_spec, _w_spec,
              _b_spec],
    out_specs=[_half_spec, _half_spec], out_shape=[_half_sds, _half_sds])

_tc_layer_pred = pl.pallas_call(
    _layer_pred_body, grid=(G,),
    in_specs=[_agg_spec, _deg_spec, _half_spec, _half_spec, _w_spec, _w_spec,
              _b_spec, _w_spec, _b_spec],
    out_specs=_row_spec,
    out_shape=jax.ShapeDtypeStruct((N, D), jnp.float32))


def kernel(nodes, edge_index, W_enc, b_enc, W_root, W_neigh, b_conv,
           W_pred, b_pred):
    pad_i = jnp.arange(PADE, dtype=jnp.int32)
    pad_src = (pad_i * 131) % N
    pad_dst = N + pad_i % (NPAD - N)
    src = jnp.concatenate([edge_index[0], pad_src]).reshape(NS, NCHT, CH)
    dst = jnp.concatenate([edge_index[1], pad_dst]).reshape(NS, NCHT, CH)
    z64 = jnp.zeros((ZR, D2), jnp.float32)
    z16 = jnp.zeros((ZR, DEGW), jnp.float32)
    ones_ch = jnp.ones((CH, DEGW), jnp.float32)

    h0lo, h0hi = _encoder(nodes, W_enc, b_enc.reshape(1, D))
    agg1, deg = _sc_layer_deg(h0lo, h0hi, src, dst, z64, z16, ones_ch)
    h1lo, h1hi = _tc_layer(agg1, deg, h0lo, h0hi, W_neigh[0], W_root[0],
                           b_conv[0].reshape(1, D))
    (agg2,) = _sc_layer(h1lo, h1hi, src, dst, z64)
    return _tc_layer_pred(agg2, deg, h1lo, h1hi, W_neigh[1], W_root[1],
                          b_conv[1].reshape(1, D), W_pred,
                          b_pred.reshape(1, D))
